# baseline (speedup 1.0000x reference)
import functools

import jax
import jax.numpy as jnp
from jax import lax
from jax.experimental import pallas as pl
from jax.experimental.pallas import tpu as pltpu
from jax.experimental.pallas import tpu_sc as plsc

BATCH = 16384
NA = 51

NC = 2
NS = 16
L = 16
NW = NC * NS

SC_ROWS = 2048
TC_ROWS = BATCH - SC_ROWS
TC_B = 1024
SC_RPW = SC_ROWS // NW
SC_GROUPS = SC_RPW // L
SC_CHUNK = SC_RPW * NA


def _log16(s):
    bits = plsc.bitcast(s, jnp.int32)
    e = lax.shift_right_arithmetic(bits, 23) - 127
    mbits = lax.bitwise_or(lax.bitwise_and(bits, 0x7FFFFF), 0x3F800000)
    m = plsc.bitcast(mbits, jnp.float32)
    big = m >= 1.41421356
    m = jnp.where(big, m * 0.5, m)
    e = e + big.astype(jnp.int32)
    ef = e.astype(jnp.float32)
    f = m - 1.0
    z = f * f
    p = jnp.float32(7.0376836292e-2)
    p = p * f + (-1.1514610310e-1)
    p = p * f + 1.1676998740e-1
    p = p * f + (-1.2420140846e-1)
    p = p * f + 1.4249322787e-1
    p = p * f + (-1.6668057665e-1)
    p = p * f + 2.0000714765e-1
    p = p * f + (-2.4999993993e-1)
    p = p * f + 3.3333331174e-1
    y = f * z * p
    y = y + ef * (-2.12194440e-4)
    y = y - 0.5 * z
    return f + y + ef * 0.693359375


def _sc_tile_body(l_hbm, t_hbm, out_hbm, l_v, t_v, acc_v):
    wid = lax.axis_index("s") * NC + lax.axis_index("c")
    base = wid * SC_CHUNK
    pltpu.sync_copy(l_hbm.at[pl.ds(base, SC_CHUNK)], l_v)
    pltpu.sync_copy(t_hbm.at[pl.ds(base, SC_CHUNK)], t_v)

    row_off = lax.iota(jnp.int32, L) * NA

    def group(g, acc):
        idx0 = row_off + g * (L * NA)
        parts = []
        for k in range(3):
            mk = None
            for j in range(k, NA, 3):
                v = plsc.load_gather(l_v, [idx0 + j])
                mk = v if mk is None else jnp.maximum(mk, v)
            parts.append(mk)
        m = jnp.maximum(jnp.maximum(parts[0], parts[1]), parts[2])

        ssum = []
        tsum = []
        dsum = []
        for k in range(3):
            sk = None
            tk = None
            dk = None
            for j in range(k, NA, 3):
                idx = idx0 + j
                lv = plsc.load_gather(l_v, [idx])
                tv = plsc.load_gather(t_v, [idx])
                ev = jnp.exp(lv - m)
                sk = ev if sk is None else sk + ev
                tk = tv if tk is None else tk + tv
                pv = tv * lv
                dk = pv if dk is None else dk + pv
            ssum.append(sk)
            tsum.append(tk)
            dsum.append(dk)
        s = ssum[0] + ssum[1] + ssum[2]
        st = tsum[0] + tsum[1] + tsum[2]
        dot = dsum[0] + dsum[1] + dsum[2]

        lse = _log16(s) + m
        return acc + (lse * st - dot)

    acc = lax.fori_loop(0, SC_GROUPS, group, jnp.zeros((L,), jnp.float32))
    acc_v[...] = acc
    pltpu.sync_copy(acc_v, out_hbm.at[wid])


def _sc_loss(l_flat, t_flat):
    k = pl.kernel(
        _sc_tile_body,
        out_type=jax.ShapeDtypeStruct((NW, L), jnp.float32),
        mesh=plsc.VectorSubcoreMesh(core_axis_name="c", subcore_axis_name="s"),
        scratch_types=[
            pltpu.VMEM((SC_CHUNK,), jnp.float32),
            pltpu.VMEM((SC_CHUNK,), jnp.float32),
            pltpu.VMEM((L,), jnp.float32),
        ],
        compiler_params=pltpu.CompilerParams(needs_layout_passes=False),
    )
    return k(l_flat, t_flat)


def _tc_block_body(l_ref, t_ref, out_ref):
    @pl.when(pl.program_id(0) == 0)
    def _():
        out_ref[0, 0] = jnp.float32(0.0)

    l = l_ref[...]
    t = t_ref[...]
    m = jnp.max(l, axis=-1, keepdims=True)
    s = jnp.sum(jnp.exp(l - m), axis=-1, keepdims=True)
    lse = jnp.log(s) + m
    st = jnp.sum(t, axis=-1, keepdims=True)
    dot = jnp.sum(t * l, axis=-1, keepdims=True)
    out_ref[0, 0] += jnp.sum(lse * st - dot)


def _tc_loss(l, t):
    skip = SC_ROWS // TC_B
    return pl.pallas_call(
        _tc_block_body,
        grid=(TC_ROWS // TC_B,),
        in_specs=[
            pl.BlockSpec((TC_B, NA), lambda i: (i + skip, 0)),
            pl.BlockSpec((TC_B, NA), lambda i: (i + skip, 0)),
        ],
        out_specs=pl.BlockSpec(
            (1, 1), lambda i: (0, 0), memory_space=pltpu.SMEM
        ),
        out_shape=jax.ShapeDtypeStruct((1, 1), jnp.float32),
    )(l, t)


def kernel(current_logits, target_distribution):
    l_sc = current_logits[:SC_ROWS].reshape(-1)
    t_sc = target_distribution[:SC_ROWS].reshape(-1)
    sc_part = _sc_loss(l_sc, t_sc)
    tc_part = _tc_loss(current_logits, target_distribution)
    return (jnp.sum(sc_part) + tc_part[0, 0]) / jnp.float32(BATCH)


# speedup vs baseline: 1.7027x; 1.7027x over previous
import functools

import jax
import jax.numpy as jnp
from jax import lax
from jax.experimental import pallas as pl
from jax.experimental.pallas import tpu as pltpu
from jax.experimental.pallas import tpu_sc as plsc

BATCH = 16384
NA = 51

NC = 2
NS = 16
L = 16
NW = NC * NS

SC_ROWS = 0
TC_ROWS = BATCH - SC_ROWS
TC_B = 1024
SC_RPW = SC_ROWS // NW
SC_GROUPS = SC_RPW // L
SC_CHUNK = SC_RPW * NA


def _log16(s):
    bits = plsc.bitcast(s, jnp.int32)
    e = lax.shift_right_arithmetic(bits, 23) - 127
    mbits = lax.bitwise_or(lax.bitwise_and(bits, 0x7FFFFF), 0x3F800000)
    m = plsc.bitcast(mbits, jnp.float32)
    big = m >= 1.41421356
    m = jnp.where(big, m * 0.5, m)
    e = e + big.astype(jnp.int32)
    ef = e.astype(jnp.float32)
    f = m - 1.0
    z = f * f
    p = jnp.float32(7.0376836292e-2)
    p = p * f + (-1.1514610310e-1)
    p = p * f + 1.1676998740e-1
    p = p * f + (-1.2420140846e-1)
    p = p * f + 1.4249322787e-1
    p = p * f + (-1.6668057665e-1)
    p = p * f + 2.0000714765e-1
    p = p * f + (-2.4999993993e-1)
    p = p * f + 3.3333331174e-1
    y = f * z * p
    y = y + ef * (-2.12194440e-4)
    y = y - 0.5 * z
    return f + y + ef * 0.693359375


def _sc_tile_body(l_hbm, t_hbm, out_hbm, l_v, t_v, acc_v):
    wid = lax.axis_index("s") * NC + lax.axis_index("c")
    base = wid * SC_CHUNK
    pltpu.sync_copy(l_hbm.at[pl.ds(base, SC_CHUNK)], l_v)
    pltpu.sync_copy(t_hbm.at[pl.ds(base, SC_CHUNK)], t_v)

    row_off = lax.iota(jnp.int32, L) * NA

    def group(g, acc):
        idx0 = row_off + g * (L * NA)
        parts = []
        for k in range(3):
            mk = None
            for j in range(k, NA, 3):
                v = plsc.load_gather(l_v, [idx0 + j])
                mk = v if mk is None else jnp.maximum(mk, v)
            parts.append(mk)
        m = jnp.maximum(jnp.maximum(parts[0], parts[1]), parts[2])

        ssum = []
        tsum = []
        dsum = []
        for k in range(3):
            sk = None
            tk = None
            dk = None
            for j in range(k, NA, 3):
                idx = idx0 + j
                lv = plsc.load_gather(l_v, [idx])
                tv = plsc.load_gather(t_v, [idx])
                ev = jnp.exp(lv - m)
                sk = ev if sk is None else sk + ev
                tk = tv if tk is None else tk + tv
                pv = tv * lv
                dk = pv if dk is None else dk + pv
            ssum.append(sk)
            tsum.append(tk)
            dsum.append(dk)
        s = ssum[0] + ssum[1] + ssum[2]
        st = tsum[0] + tsum[1] + tsum[2]
        dot = dsum[0] + dsum[1] + dsum[2]

        lse = _log16(s) + m
        return acc + (lse * st - dot)

    acc = lax.fori_loop(0, SC_GROUPS, group, jnp.zeros((L,), jnp.float32))
    acc_v[...] = acc
    pltpu.sync_copy(acc_v, out_hbm.at[wid])


def _sc_loss(l_flat, t_flat):
    k = pl.kernel(
        _sc_tile_body,
        out_type=jax.ShapeDtypeStruct((NW, L), jnp.float32),
        mesh=plsc.VectorSubcoreMesh(core_axis_name="c", subcore_axis_name="s"),
        scratch_types=[
            pltpu.VMEM((SC_CHUNK,), jnp.float32),
            pltpu.VMEM((SC_CHUNK,), jnp.float32),
            pltpu.VMEM((L,), jnp.float32),
        ],
        compiler_params=pltpu.CompilerParams(needs_layout_passes=False),
    )
    return k(l_flat, t_flat)


def _tc_block_body(l_ref, t_ref, out_ref):
    @pl.when(pl.program_id(0) == 0)
    def _():
        out_ref[0, 0] = jnp.float32(0.0)

    l = l_ref[...]
    t = t_ref[...]
    m = jnp.max(l, axis=-1, keepdims=True)
    s = jnp.sum(jnp.exp(l - m), axis=-1, keepdims=True)
    lse = jnp.log(s) + m
    st = jnp.sum(t, axis=-1, keepdims=True)
    dot = jnp.sum(t * l, axis=-1, keepdims=True)
    out_ref[0, 0] += jnp.sum(lse * st - dot)


def _tc_loss(l, t):
    skip = SC_ROWS // TC_B
    return pl.pallas_call(
        _tc_block_body,
        grid=(TC_ROWS // TC_B,),
        in_specs=[
            pl.BlockSpec((TC_B, NA), lambda i: (i + skip, 0)),
            pl.BlockSpec((TC_B, NA), lambda i: (i + skip, 0)),
        ],
        out_specs=pl.BlockSpec(
            (1, 1), lambda i: (0, 0), memory_space=pltpu.SMEM
        ),
        out_shape=jax.ShapeDtypeStruct((1, 1), jnp.float32),
    )(l, t)


def kernel(current_logits, target_distribution):
    tc_part = _tc_loss(current_logits, target_distribution)
    return tc_part[0, 0] / jnp.float32(BATCH)
